# Initial kernel scaffold; baseline (speedup 1.0000x reference)
import jax
import jax.numpy as jnp
from jax.experimental import pallas as pl


def kernel(x, edge_index, edge_attr, W1, b1, W2, b2, Wih_f, Whh_f, bih_f, bhh_f, Wih_b, Whh_b, bih_b, bhh_b, Wl, bl):
    raise NotImplementedError("write your pallas kernel here")


# speedup vs baseline: 21.8115x; 21.8115x over previous
import dataclasses

import jax
import jax.numpy as jnp
from jax.experimental import pallas as pl
from jax.experimental.pallas import tpu as pltpu
from jax.experimental.pallas import tpu_sc as plsc

N = 10000
E = 320000
H = 128
LH = 64
NP = 10240
NC = 2
NS = 16
NW = NC * NS
EPW = E // NW
CD = 2000
CE = 200
SLICE = NP // NS

_f32 = jnp.float32
_i32 = jnp.int32


def _vector_mesh():
    return plsc.VectorSubcoreMesh(core_axis_name="c", subcore_axis_name="s")


def _sc_params():
    cp = pltpu.CompilerParams()
    if "needs_layout_passes" in pltpu.CompilerParams.__dataclass_fields__:
        cp = dataclasses.replace(cp, needs_layout_passes=False)
    return cp


def _deg_body(col_hbm, ea_hbm, out_hbm, colv, eav, degv, tmpv, accv, shared):
    cid = jax.lax.axis_index("c")
    sid = jax.lax.axis_index("s")
    wid = cid * NS + sid

    @pl.loop(0, NP, step=16)
    def _zero(i):
        degv[pl.ds(i, 16)] = jnp.zeros((16,), _f32)

    @pl.loop(0, EPW, step=CD)
    def _chunk(cb):
        base = pl.multiple_of(wid * EPW + cb, 8)
        pltpu.sync_copy(col_hbm.at[pl.ds(base, CD)], colv)
        pltpu.sync_copy(ea_hbm.at[pl.ds(base, CD)], eav)

        @pl.loop(0, CD, step=16)
        def _vec(j):
            idx = colv[pl.ds(j, 16)]
            val = eav[pl.ds(j, 16)]
            plsc.addupdate_scatter(degv, [idx], val)

    pltpu.sync_copy(degv, shared.at[sid])
    plsc.subcore_barrier()

    off = sid * SLICE

    @pl.loop(0, SLICE, step=16)
    def _zacc(i):
        accv[pl.ds(i, 16)] = jnp.zeros((16,), _f32)

    for w in range(NS):
        pltpu.sync_copy(shared.at[w, pl.ds(off, SLICE)], tmpv)

        @pl.loop(0, SLICE, step=16)
        def _acc(j):
            accv[pl.ds(j, 16)] += tmpv[pl.ds(j, 16)]

    pltpu.sync_copy(accv, out_hbm.at[cid, pl.ds(off, SLICE)])


def _deg_call(col, ea):
    kern = pl.kernel(
        _deg_body,
        out_type=jax.ShapeDtypeStruct((NC, NP), _f32),
        mesh=_vector_mesh(),
        compiler_params=_sc_params(),
        scratch_types=[
            pltpu.VMEM((CD,), _i32),
            pltpu.VMEM((CD,), _f32),
            pltpu.VMEM((NP,), _f32),
            pltpu.VMEM((SLICE,), _f32),
            pltpu.VMEM((SLICE,), _f32),
            pltpu.VMEM_SHARED((NS, NP), _f32),
        ],
    )
    return kern(col, ea)


def _edge_body(y_hbm, row_hbm, col_hbm, ea_hbm, out_hbm,
               rowiv, coliv, eav, rowsv, zbuf, oshared, gsem):
    cid = jax.lax.axis_index("c")
    sid = jax.lax.axis_index("s")
    wid = cid * NS + sid

    @pl.loop(0, 128)
    def _zb(i):
        for k in range(H // 16):
            zbuf[i, pl.ds(k * 16, 16)] = jnp.zeros((16,), _f32)

    for k in range(SLICE // 128):
        r0 = sid * SLICE + k * 128
        pltpu.sync_copy(zbuf, oshared.at[pl.ds(r0, 128), :])
    plsc.subcore_barrier()

    @pl.loop(0, EPW, step=CE)
    def _chunk(cb):
        base = pl.multiple_of(wid * EPW + cb, 8)
        pltpu.sync_copy(row_hbm.at[pl.ds(base, CE)], rowiv)
        pltpu.sync_copy(col_hbm.at[pl.ds(base, CE)], coliv)
        pltpu.sync_copy(ea_hbm.at[pl.ds(base, CE)], eav)
        pltpu.async_copy(y_hbm.at[rowiv], rowsv, gsem).wait()

        @pl.loop(0, CE)
        def _scale(e):
            s = plsc.load_gather(eav, [jnp.full((16,), e, _i32)])
            for k in range(H // 16):
                sl = pl.ds(k * 16, 16)
                rowsv[e, sl] = rowsv[e, sl] * s

        pltpu.sync_copy(rowsv, oshared.at[coliv], add=True)

    plsc.subcore_barrier()
    for k in range(SLICE // 128):
        r0 = sid * SLICE + k * 128
        pltpu.sync_copy(oshared.at[pl.ds(r0, 128), :],
                        out_hbm.at[cid, pl.ds(r0, 128), :])


def _edge_call(y, row, col, ea):
    kern = pl.kernel(
        _edge_body,
        out_type=jax.ShapeDtypeStruct((NC, NP, H), _f32),
        mesh=_vector_mesh(),
        compiler_params=_sc_params(),
        scratch_types=[
            pltpu.VMEM((CE,), _i32),
            pltpu.VMEM((CE,), _i32),
            pltpu.VMEM((CE,), _f32),
            pltpu.VMEM((CE, H), _f32),
            pltpu.VMEM((128, H), _f32),
            pltpu.VMEM_SHARED((NP, H), _f32),
            pltpu.SemaphoreType.DMA,
        ],
    )
    return kern(y, row, col, ea)


def _dinv_body(deg_ref, out_ref):
    dd = deg_ref[0:1, :] + deg_ref[1:2, :]
    out_ref[...] = jnp.where(dd > 0, jax.lax.rsqrt(jnp.where(dd > 0, dd, 1.0)), 0.0)


def _dinv_call(deg2):
    return pl.pallas_call(
        _dinv_body,
        out_shape=jax.ShapeDtypeStruct((1, NP), _f32),
    )(deg2)


def _mm_scale_body(x_ref, w_ref, dinv_ref, out_ref):
    xw = jnp.dot(x_ref[...], w_ref[...], preferred_element_type=_f32)
    out_ref[...] = xw * dinv_ref[...]


def _mm_scale_call(x, w, dinv_col):
    return pl.pallas_call(
        _mm_scale_body,
        out_shape=jax.ShapeDtypeStruct((N, H), _f32),
    )(x, w, dinv_col)


def _comb1_body(p0_ref, p1_ref, dinv_ref, b_ref, w_ref, out_ref):
    h = jax.nn.relu(dinv_ref[...] * (p0_ref[...] + p1_ref[...]) + b_ref[...])
    out_ref[...] = jnp.dot(h, w_ref[...], preferred_element_type=_f32) * dinv_ref[...]


def _comb1_call(p0, p1, dinv_col, b1, w2):
    return pl.pallas_call(
        _comb1_body,
        out_shape=jax.ShapeDtypeStruct((N, H), _f32),
    )(p0, p1, dinv_col, b1, w2)


def _comb2_body(p0_ref, p1_ref, dinv_ref, b_ref, wf_ref, wb_ref, bf_ref, bb_ref,
                gxf_ref, gxb_ref):
    h = jax.nn.relu(dinv_ref[...] * (p0_ref[...] + p1_ref[...]) + b_ref[...])
    gxf_ref[...] = jnp.dot(h, wf_ref[...], preferred_element_type=_f32) + bf_ref[...]
    gxb_ref[...] = jnp.dot(h, wb_ref[...], preferred_element_type=_f32) + bb_ref[...]


def _comb2_call(p0, p1, dinv_col, b2, wf, wb, bf, bb):
    return pl.pallas_call(
        _comb2_body,
        out_shape=(jax.ShapeDtypeStruct((N, 4 * LH), _f32),
                   jax.ShapeDtypeStruct((N, 4 * LH), _f32)),
    )(p0, p1, dinv_col, b2, wf, wb, bf, bb)


def _lstm_body(gx_ref, wc_ref, hs_ref):
    wc = wc_ref[...]

    def step(t, carry):
        h, c = carry
        z = jnp.dot(h, wc, preferred_element_type=_f32) + gx_ref[pl.ds(t, 1), :]
        gi = jax.nn.sigmoid(z[:, 0:128])
        gf = jax.nn.sigmoid(z[:, 128:256])
        gg = jnp.tanh(z[:, 256:384])
        go = jax.nn.sigmoid(z[:, 384:512])
        c = gf * c + gi * gg
        h = go * jnp.tanh(c)
        hs_ref[pl.ds(t, 1), 0:LH] = h[:, 0:LH]
        hs_ref[pl.ds(N - 1 - t, 1), LH:2 * LH] = h[:, LH:2 * LH]
        return h, c

    h0 = jnp.zeros((1, 2 * LH), _f32)
    c0 = jnp.zeros((1, 2 * LH), _f32)
    jax.lax.fori_loop(0, N, step, (h0, c0))


def _lstm_call(gxcat, wc):
    return pl.pallas_call(
        _lstm_body,
        out_shape=jax.ShapeDtypeStruct((N, 2 * LH), _f32),
    )(gxcat, wc)


def _head_body(hs_ref, wl_ref, bl_ref, out_ref):
    out_ref[...] = jax.nn.sigmoid(
        jnp.dot(hs_ref[...], wl_ref[...], preferred_element_type=_f32)
        + bl_ref[...])


def _head_call(hs, wl, bl):
    return pl.pallas_call(
        _head_body,
        out_shape=jax.ShapeDtypeStruct((N, 1), _f32),
    )(hs, wl, bl)


def kernel(x, edge_index, edge_attr, W1, b1, W2, b2,
           Wih_f, Whh_f, bih_f, bhh_f, Wih_b, Whh_b, bih_b, bhh_b,
           Wl, bl):
    row = edge_index[0]
    col = edge_index[1]

    deg2 = _deg_call(col, edge_attr)
    dinv_col = _dinv_call(deg2).reshape(NP, 1)[:N]

    y1 = _mm_scale_call(x, W1, dinv_col)
    parts1 = _edge_call(y1, row, col, edge_attr)
    y2 = _comb1_call(parts1[0, :N], parts1[1, :N], dinv_col,
                     b1.reshape(1, H), W2)

    parts2 = _edge_call(y2, row, col, edge_attr)
    bf = (bih_f + bhh_f).reshape(1, 4 * LH)
    bb = (bih_b + bhh_b).reshape(1, 4 * LH)
    gxf, gxb = _comb2_call(parts2[0, :N], parts2[1, :N], dinv_col,
                           b2.reshape(1, H), Wih_f.T, Wih_b.T, bf, bb)

    gxbr = gxb[::-1]
    gxcat = jnp.concatenate(
        [jnp.concatenate([gxf[:, g * LH:(g + 1) * LH],
                          gxbr[:, g * LH:(g + 1) * LH]], axis=1)
         for g in range(4)], axis=1)

    wf = Whh_f.T
    wb = Whh_b.T
    zz = jnp.zeros((LH, LH), _f32)
    top = jnp.concatenate(
        [jnp.concatenate([wf[:, g * LH:(g + 1) * LH], zz], axis=1)
         for g in range(4)], axis=1)
    bot = jnp.concatenate(
        [jnp.concatenate([zz, wb[:, g * LH:(g + 1) * LH]], axis=1)
         for g in range(4)], axis=1)
    wc = jnp.concatenate([top, bot], axis=0)

    hs = _lstm_call(gxcat, wc)
    out = _head_call(hs, Wl, bl.reshape(1, 1))
    return out.T


# speedup vs baseline: 25.5880x; 1.1731x over previous
import dataclasses

import jax
import jax.numpy as jnp
from jax.experimental import pallas as pl
from jax.experimental.pallas import tpu as pltpu
from jax.experimental.pallas import tpu_sc as plsc

N = 10000
E = 320000
H = 128
LH = 64
NP = 10240
NC = 2
NS = 16
NW = NC * NS
EPW = E // NW
CD = 2000
CE = 200
SLICE = NP // NS

_f32 = jnp.float32
_i32 = jnp.int32


def _vector_mesh():
    return plsc.VectorSubcoreMesh(core_axis_name="c", subcore_axis_name="s")


def _sc_params():
    cp = pltpu.CompilerParams()
    if "needs_layout_passes" in pltpu.CompilerParams.__dataclass_fields__:
        cp = dataclasses.replace(cp, needs_layout_passes=False)
    return cp


def _deg_body(col_hbm, ea_hbm, out_hbm, colv, eav, degv, tmpv, accv, shared):
    cid = jax.lax.axis_index("c")
    sid = jax.lax.axis_index("s")
    wid = cid * NS + sid

    @pl.loop(0, NP, step=16)
    def _zero(i):
        degv[pl.ds(i, 16)] = jnp.zeros((16,), _f32)

    @pl.loop(0, EPW, step=CD)
    def _chunk(cb):
        base = pl.multiple_of(wid * EPW + cb, 8)
        pltpu.sync_copy(col_hbm.at[pl.ds(base, CD)], colv)
        pltpu.sync_copy(ea_hbm.at[pl.ds(base, CD)], eav)

        @pl.loop(0, CD, step=16)
        def _vec(j):
            idx = colv[pl.ds(j, 16)]
            val = eav[pl.ds(j, 16)]
            plsc.addupdate_scatter(degv, [idx], val)

    pltpu.sync_copy(degv, shared.at[sid])
    plsc.subcore_barrier()

    off = sid * SLICE

    @pl.loop(0, SLICE, step=16)
    def _zacc(i):
        accv[pl.ds(i, 16)] = jnp.zeros((16,), _f32)

    for w in range(NS):
        pltpu.sync_copy(shared.at[w, pl.ds(off, SLICE)], tmpv)

        @pl.loop(0, SLICE, step=16)
        def _acc(j):
            accv[pl.ds(j, 16)] += tmpv[pl.ds(j, 16)]

    pltpu.sync_copy(accv, out_hbm.at[cid, pl.ds(off, SLICE)])


def _deg_call(col, ea):
    kern = pl.kernel(
        _deg_body,
        out_type=jax.ShapeDtypeStruct((NC, NP), _f32),
        mesh=_vector_mesh(),
        compiler_params=_sc_params(),
        scratch_types=[
            pltpu.VMEM((CD,), _i32),
            pltpu.VMEM((CD,), _f32),
            pltpu.VMEM((NP,), _f32),
            pltpu.VMEM((SLICE,), _f32),
            pltpu.VMEM((SLICE,), _f32),
            pltpu.VMEM_SHARED((NS, NP), _f32),
        ],
    )
    return kern(col, ea)


def _edge_body(y_hbm, row_hbm, col_hbm, ea_hbm, out_hbm,
               rowiv, coliv, eav, rowsv, zbuf, oshared, gsem):
    cid = jax.lax.axis_index("c")
    sid = jax.lax.axis_index("s")
    wid = cid * NS + sid

    @pl.loop(0, 128)
    def _zb(i):
        for k in range(H // 16):
            zbuf[i, pl.ds(k * 16, 16)] = jnp.zeros((16,), _f32)

    for k in range(SLICE // 128):
        r0 = sid * SLICE + k * 128
        pltpu.sync_copy(zbuf, oshared.at[pl.ds(r0, 128), :])
    plsc.subcore_barrier()

    @pl.loop(0, EPW, step=CE)
    def _chunk(cb):
        base = pl.multiple_of(wid * EPW + cb, 8)
        pltpu.sync_copy(row_hbm.at[pl.ds(base, CE)], rowiv)
        pltpu.sync_copy(col_hbm.at[pl.ds(base, CE)], coliv)
        pltpu.sync_copy(ea_hbm.at[pl.ds(base, CE)], eav)
        pltpu.async_copy(y_hbm.at[rowiv], rowsv, gsem).wait()

        @pl.loop(0, CE)
        def _scale(e):
            s = plsc.load_gather(eav, [jnp.full((16,), e, _i32)])
            for k in range(H // 16):
                sl = pl.ds(k * 16, 16)
                rowsv[e, sl] = rowsv[e, sl] * s

        pltpu.sync_copy(rowsv, oshared.at[coliv], add=True)

    plsc.subcore_barrier()
    for k in range(SLICE // 128):
        r0 = sid * SLICE + k * 128
        pltpu.sync_copy(oshared.at[pl.ds(r0, 128), :],
                        out_hbm.at[cid, pl.ds(r0, 128), :])


def _edge_call(y, row, col, ea):
    kern = pl.kernel(
        _edge_body,
        out_type=jax.ShapeDtypeStruct((NC, NP, H), _f32),
        mesh=_vector_mesh(),
        compiler_params=_sc_params(),
        scratch_types=[
            pltpu.VMEM((CE,), _i32),
            pltpu.VMEM((CE,), _i32),
            pltpu.VMEM((CE,), _f32),
            pltpu.VMEM((CE, H), _f32),
            pltpu.VMEM((128, H), _f32),
            pltpu.VMEM_SHARED((NP, H), _f32),
            pltpu.SemaphoreType.DMA,
        ],
    )
    return kern(y, row, col, ea)


def _dinv_body(deg_ref, out_ref):
    dd = deg_ref[0:1, :] + deg_ref[1:2, :]
    out_ref[...] = jnp.where(dd > 0, jax.lax.rsqrt(jnp.where(dd > 0, dd, 1.0)), 0.0)


def _dinv_call(deg2):
    return pl.pallas_call(
        _dinv_body,
        out_shape=jax.ShapeDtypeStruct((1, NP), _f32),
    )(deg2)


def _mm_scale_body(x_ref, w_ref, dinv_ref, out_ref):
    xw = jnp.dot(x_ref[...], w_ref[...], preferred_element_type=_f32)
    out_ref[...] = xw * dinv_ref[...]


def _mm_scale_call(x, w, dinv_col):
    return pl.pallas_call(
        _mm_scale_body,
        out_shape=jax.ShapeDtypeStruct((N, H), _f32),
    )(x, w, dinv_col)


def _comb1_body(p0_ref, p1_ref, dinv_ref, b_ref, w_ref, out_ref):
    h = jax.nn.relu(dinv_ref[...] * (p0_ref[...] + p1_ref[...]) + b_ref[...])
    out_ref[...] = jnp.dot(h, w_ref[...], preferred_element_type=_f32) * dinv_ref[...]


def _comb1_call(p0, p1, dinv_col, b1, w2):
    return pl.pallas_call(
        _comb1_body,
        out_shape=jax.ShapeDtypeStruct((N, H), _f32),
    )(p0, p1, dinv_col, b1, w2)


def _comb2_body(p0_ref, p1_ref, dinv_ref, b_ref, wf_ref, wb_ref, bf_ref, bb_ref,
                gxf_ref, gxb_ref):
    h = jax.nn.relu(dinv_ref[...] * (p0_ref[...] + p1_ref[...]) + b_ref[...])
    gxf_ref[...] = jnp.dot(h, wf_ref[...], preferred_element_type=_f32) + bf_ref[...]
    gxb_ref[...] = jnp.dot(h, wb_ref[...], preferred_element_type=_f32) + bb_ref[...]


def _comb2_call(p0, p1, dinv_col, b2, wf, wb, bf, bb):
    return pl.pallas_call(
        _comb2_body,
        out_shape=(jax.ShapeDtypeStruct((N, 4 * LH), _f32),
                   jax.ShapeDtypeStruct((N, 4 * LH), _f32)),
    )(p0, p1, dinv_col, b2, wf, wb, bf, bb)


def _lstm_body(gx_ref, wc_ref, hs_ref):
    wc = wc_ref[...]

    def step(t, carry):
        h, c = carry
        ht = h.reshape(2 * LH, 1)
        z = (jnp.sum(ht * wc, axis=0, keepdims=True)
             + gx_ref[pl.ds(t, 1), :])
        gi = jax.nn.sigmoid(z[:, 0:128])
        gf = jax.nn.sigmoid(z[:, 128:256])
        gg = jnp.tanh(z[:, 256:384])
        go = jax.nn.sigmoid(z[:, 384:512])
        c = gf * c + gi * gg
        h = go * jnp.tanh(c)
        hs_ref[pl.ds(t, 1), 0:LH] = h[:, 0:LH]
        hs_ref[pl.ds(N - 1 - t, 1), LH:2 * LH] = h[:, LH:2 * LH]
        return h, c

    h0 = jnp.zeros((1, 2 * LH), _f32)
    c0 = jnp.zeros((1, 2 * LH), _f32)
    jax.lax.fori_loop(0, N, step, (h0, c0))


def _lstm_call(gxcat, wc):
    return pl.pallas_call(
        _lstm_body,
        out_shape=jax.ShapeDtypeStruct((N, 2 * LH), _f32),
    )(gxcat, wc)


def _head_body(hs_ref, wl_ref, bl_ref, out_ref):
    out_ref[...] = jax.nn.sigmoid(
        jnp.dot(hs_ref[...], wl_ref[...], preferred_element_type=_f32)
        + bl_ref[...])


def _head_call(hs, wl, bl):
    return pl.pallas_call(
        _head_body,
        out_shape=jax.ShapeDtypeStruct((N, 1), _f32),
    )(hs, wl, bl)


def kernel(x, edge_index, edge_attr, W1, b1, W2, b2,
           Wih_f, Whh_f, bih_f, bhh_f, Wih_b, Whh_b, bih_b, bhh_b,
           Wl, bl):
    row = edge_index[0]
    col = edge_index[1]

    deg2 = _deg_call(col, edge_attr)
    dinv_col = _dinv_call(deg2).reshape(NP, 1)[:N]

    y1 = _mm_scale_call(x, W1, dinv_col)
    parts1 = _edge_call(y1, row, col, edge_attr)
    y2 = _comb1_call(parts1[0, :N], parts1[1, :N], dinv_col,
                     b1.reshape(1, H), W2)

    parts2 = _edge_call(y2, row, col, edge_attr)
    bf = (bih_f + bhh_f).reshape(1, 4 * LH)
    bb = (bih_b + bhh_b).reshape(1, 4 * LH)
    gxf, gxb = _comb2_call(parts2[0, :N], parts2[1, :N], dinv_col,
                           b2.reshape(1, H), Wih_f.T, Wih_b.T, bf, bb)

    gxbr = gxb[::-1]
    gxcat = jnp.concatenate(
        [jnp.concatenate([gxf[:, g * LH:(g + 1) * LH],
                          gxbr[:, g * LH:(g + 1) * LH]], axis=1)
         for g in range(4)], axis=1)

    wf = Whh_f.T
    wb = Whh_b.T
    zz = jnp.zeros((LH, LH), _f32)
    top = jnp.concatenate(
        [jnp.concatenate([wf[:, g * LH:(g + 1) * LH], zz], axis=1)
         for g in range(4)], axis=1)
    bot = jnp.concatenate(
        [jnp.concatenate([zz, wb[:, g * LH:(g + 1) * LH]], axis=1)
         for g in range(4)], axis=1)
    wc = jnp.concatenate([top, bot], axis=0)

    hs = _lstm_call(gxcat, wc)
    out = _head_call(hs, Wl, bl.reshape(1, 1))
    return out.T
